# Initial kernel scaffold; baseline (speedup 1.0000x reference)
import jax
import jax.numpy as jnp
from jax.experimental import pallas as pl


def kernel(indices, tables, proj_w, proj_b):
    raise NotImplementedError("write your pallas kernel here")


# speedup vs baseline: 7.8751x; 7.8751x over previous
import functools

import jax
import jax.numpy as jnp
from jax import lax
from jax.experimental import pallas as pl
from jax.experimental.pallas import tpu as pltpu
from jax.experimental.pallas import tpu_sc as plsc

_NUM_FIELDS = 26
_VOCAB = 100000
_EMBED = 16
_HIDDEN = 256
_BATCH = 16384

_NC = 2
_NS = 16
_NW = _NC * _NS


def _gather_sc(flat_idx, flat_table):
    R = flat_idx.shape[0]
    per_w = R // _NW
    n_chunks = 4
    ch = per_w // n_chunks

    mesh = plsc.VectorSubcoreMesh(core_axis_name="c", subcore_axis_name="s")

    @functools.partial(
        pl.kernel,
        mesh=mesh,
        compiler_params=pltpu.CompilerParams(use_tc_tiling_on_sc=False),
        out_type=jax.ShapeDtypeStruct((R, _EMBED), jnp.float32),
        scratch_types=[
            pltpu.VMEM((ch,), jnp.int32),
            pltpu.VMEM((ch, _EMBED), jnp.float32),
            pltpu.SemaphoreType.DMA,
        ],
    )
    def k(idx_hbm, tab_hbm, out_hbm, idx_v, rows_v, sem):
        wid = lax.axis_index("s") * _NC + lax.axis_index("c")
        base = wid * per_w
        for j in range(n_chunks):
            off = base + j * ch
            pltpu.sync_copy(idx_hbm.at[pl.ds(off, ch)], idx_v)
            pltpu.async_copy(tab_hbm.at[idx_v], rows_v, sem).wait()
            pltpu.sync_copy(rows_v, out_hbm.at[pl.ds(off, ch)])

    return k(flat_idx, flat_table)


def _project_tc(feats, w, b):
    B, K = feats.shape
    H = w.shape[1]
    blk = 2048

    def mm(f_ref, w_ref, b_ref, o_ref):
        o_ref[...] = (
            jnp.dot(f_ref[...], w_ref[...], preferred_element_type=jnp.float32)
            + b_ref[...]
        )

    return pl.pallas_call(
        mm,
        grid=(B // blk,),
        in_specs=[
            pl.BlockSpec((blk, K), lambda i: (i, 0)),
            pl.BlockSpec((K, H), lambda i: (0, 0)),
            pl.BlockSpec((1, H), lambda i: (0, 0)),
        ],
        out_specs=pl.BlockSpec((blk, H), lambda i: (i, 0)),
        out_shape=jax.ShapeDtypeStruct((B, H), jnp.float32),
    )(feats, w, b.reshape(1, H))


def kernel(indices, tables, proj_w, proj_b):
    F, B = indices.shape
    V, E = tables.shape[1], tables.shape[2]
    offs = (jnp.arange(F, dtype=jnp.int32) * V)[:, None]
    flat_idx = (indices + offs).T.reshape(-1)
    flat_table = tables.reshape(F * V, E)
    feats = _gather_sc(flat_idx, flat_table).reshape(B, F * E)
    return _project_tc(feats, proj_w, proj_b)
